# Initial kernel scaffold; baseline (speedup 1.0000x reference)
import jax
import jax.numpy as jnp
from jax.experimental import pallas as pl


def kernel(x, adj, W1, b1, W2, b2, W3, b3):
    raise NotImplementedError("write your pallas kernel here")


# speedup vs baseline: 7.1915x; 7.1915x over previous
import jax
import jax.numpy as jnp
from jax import lax
from jax.experimental import pallas as pl
from jax.experimental.pallas import tpu as pltpu
from jax.experimental.pallas import tpu_sc as plsc

N_NODES = 10000
N_EDGES = 320000
F = 128
CHUNK = 128
NCHUNKS = N_EDGES // CHUNK
NW = 32
CHUNKS_PER_W = (NCHUNKS + NW - 1) // NW
N_PAD = 10240
ROWS_PER_TEC = N_PAD // 16


def _spmm_body(src_hbm, dst_hbm, table_hbm, out_hbm,
               srcbuf, dstbuf, rowsbuf, zbuf, acc, sem):
    c = lax.axis_index("c")
    s = lax.axis_index("s")
    wid = c * 16 + s

    z16 = jnp.zeros((16,), jnp.float32)

    def zrow(i, carry):
        for j in range(8):
            zbuf[i, pl.ds(j * 16, 16)] = z16
        return carry

    lax.fori_loop(0, 128, zrow, 0)
    r0 = s * ROWS_PER_TEC
    for t in range(ROWS_PER_TEC // 128):
        pltpu.sync_copy(zbuf.at[...], acc.at[pl.ds(r0 + t * 128, 128)])
    plsc.subcore_barrier()

    def body(j, carry):
        cid = j * NW + wid

        @pl.when(cid < NCHUNKS)
        def _():
            off = cid * CHUNK
            pltpu.sync_copy(src_hbm.at[pl.ds(off, CHUNK)], srcbuf.at[0])
            pltpu.sync_copy(dst_hbm.at[pl.ds(off, CHUNK)], dstbuf.at[0])
            pltpu.async_copy(table_hbm.at[srcbuf.at[0]], rowsbuf, sem).wait()
            pltpu.sync_copy(rowsbuf, acc.at[dstbuf.at[0]], add=True)

        return carry

    lax.fori_loop(0, CHUNKS_PER_W, body, 0)

    plsc.subcore_barrier()
    pltpu.sync_copy(acc.at[pl.ds(r0, ROWS_PER_TEC)],
                    out_hbm.at[c, pl.ds(r0, ROWS_PER_TEC)])


import functools


@functools.cache
def _make_spmm():
    return pl.kernel(
        _spmm_body,
        out_type=jax.ShapeDtypeStruct((2, N_PAD, F), jnp.float32),
        mesh=plsc.VectorSubcoreMesh(core_axis_name="c", subcore_axis_name="s"),
        scratch_types=[
            pltpu.VMEM((1, CHUNK), jnp.int32),
            pltpu.VMEM((1, CHUNK), jnp.int32),
            pltpu.VMEM((CHUNK, F), jnp.float32),
            pltpu.VMEM((128, F), jnp.float32),
            pltpu.VMEM_SHARED((N_PAD, F), jnp.float32),
            pltpu.SemaphoreType.DMA,
        ],
    )


def _spmm(src, dst, table):
    return _make_spmm()(src, dst, table)


R = 1000


def _mlp1_body(p_ref, W1_ref, b1_ref, W2_ref, out_ref):
    h = p_ref[0] + p_ref[1]
    h1 = jnp.dot(h, W1_ref[...], preferred_element_type=jnp.float32)
    h1 = jnp.maximum(h1 + b1_ref[...], 0.0)
    out_ref[...] = jnp.dot(h1, W2_ref[...], preferred_element_type=jnp.float32)


def _mlp1(p, W1, b1, W2):
    return pl.pallas_call(
        _mlp1_body,
        grid=(N_NODES // R,),
        in_specs=[
            pl.BlockSpec((2, R, F), lambda i: (0, i, 0)),
            pl.BlockSpec((F, 256), lambda i: (0, 0)),
            pl.BlockSpec((1, 256), lambda i: (0, 0)),
            pl.BlockSpec((256, F), lambda i: (0, 0)),
        ],
        out_specs=pl.BlockSpec((R, F), lambda i: (i, 0)),
        out_shape=jax.ShapeDtypeStruct((N_NODES, F), jnp.float32),
    )(p, W1, b1.reshape(1, 256), W2)


def _mlp2_body(p_ref, b2_ref, W3_ref, out_ref):
    h = jnp.maximum(p_ref[0] + p_ref[1] + b2_ref[...], 0.0)
    out_ref[...] = jnp.dot(h, W3_ref[...], preferred_element_type=jnp.float32)


def _mlp2(p, b2, W3):
    return pl.pallas_call(
        _mlp2_body,
        grid=(N_NODES // R,),
        in_specs=[
            pl.BlockSpec((2, R, F), lambda i: (0, i, 0)),
            pl.BlockSpec((1, F), lambda i: (0, 0)),
            pl.BlockSpec((F, F), lambda i: (0, 0)),
        ],
        out_specs=pl.BlockSpec((R, F), lambda i: (i, 0)),
        out_shape=jax.ShapeDtypeStruct((N_NODES, F), jnp.float32),
    )(p, b2.reshape(1, F), W3)


def _bias_body(p_ref, b3_ref, out_ref):
    out_ref[...] = p_ref[0] + p_ref[1] + b3_ref[...]


def _bias(p, b3):
    return pl.pallas_call(
        _bias_body,
        grid=(N_NODES // R,),
        in_specs=[
            pl.BlockSpec((2, R, F), lambda i: (0, i, 0)),
            pl.BlockSpec((1, F), lambda i: (0, 0)),
        ],
        out_specs=pl.BlockSpec((R, F), lambda i: (i, 0)),
        out_shape=jax.ShapeDtypeStruct((N_NODES, F), jnp.float32),
    )(p, b3.reshape(1, F))


def kernel(x, adj, W1, b1, W2, b2, W3, b3):
    src = adj[0]
    dst = adj[1]
    aggx = _spmm(src, dst, x)
    s2 = _mlp1(aggx, W1, b1, W2)
    agg2 = _spmm(src, dst, s2)
    s3 = _mlp2(agg2, b2, W3)
    agg3 = _spmm(src, dst, s3)
    return _bias(agg3, b3)
